# baseline (speedup 1.0000x reference)
import functools

import jax
import jax.numpy as jnp
from jax import lax
from jax.experimental import pallas as pl
from jax.experimental.pallas import tpu as pltpu
from jax.experimental.pallas import tpu_sc as plsc

_NC = 2
_NS = 16
_L = 16
_NW = _NC * _NS
_B = 4096
_F = 64
_N = 100000
_BPW = _B // _NW
_G = _BPW // _L

_SLAB = 128
_NSLAB = _N // _SLAB
_NL = _N // 2
_TAIL = _N - _NSLAB * _SLAB
_TAIL_L = _TAIL // 2
_TOT_SLABS = 2 * _NSLAB
_SPW = -(-_TOT_SLABS // _NW)


def _transpose_body(uwt_hbm, iwt_hbm, tail_u, tail_i, ou_hbm, oi_hbm,
                    slab, block, sem_in, sem_out):
    wid = lax.axis_index("s") * _NC + lax.axis_index("c")

    lanes = lax.iota(jnp.int32, _L)
    lin_perm = lanes >> 1
    col_perm = (lanes & 1) * jnp.int32(_F)

    def do_slab(k, carry):
        s = wid + k * _NW

        @pl.when(s < _TOT_SLABS)
        def _():
            is_item = s >= _NSLAB
            ib = jnp.where(is_item, s - _NSLAB, s)

            @pl.when(jnp.logical_not(is_item))
            def _():
                pltpu.async_copy(
                    uwt_hbm.at[:, pl.ds(ib * _SLAB, _SLAB)],
                    slab, sem_in).wait()

            @pl.when(is_item)
            def _():
                pltpu.async_copy(
                    iwt_hbm.at[:, pl.ds(ib * _SLAB, _SLAB)],
                    slab, sem_in).wait()

            for f in range(_F):
                for cb in range(_SLAB // _L):
                    v = slab[f, pl.ds(cb * _L, _L)]
                    lin = lin_perm + jnp.int32(cb * (_L // 2))
                    col = col_perm + jnp.int32(f)
                    plsc.store_scatter(block, [lin, col], v)

            @pl.when(jnp.logical_not(is_item))
            def _():
                pltpu.async_copy(
                    block,
                    ou_hbm.at[pl.ds(ib * (_SLAB // 2), _SLAB // 2)],
                    sem_out).wait()

            @pl.when(is_item)
            def _():
                pltpu.async_copy(
                    block,
                    oi_hbm.at[pl.ds(ib * (_SLAB // 2), _SLAB // 2)],
                    sem_out).wait()
        return carry

    lax.fori_loop(0, _SPW, do_slab, 0)

    @pl.when(wid == 0)
    def _():
        pltpu.sync_copy(tail_u, ou_hbm.at[pl.ds(_NL - _TAIL_L, _TAIL_L)])

    @pl.when(wid == 1)
    def _():
        pltpu.sync_copy(tail_i, oi_hbm.at[pl.ds(_NL - _TAIL_L, _TAIL_L)])


def _gather_body(user_hbm, pos_hbm, neg_hbm, tu_hbm, ti_hbm, out_hbm,
                 vidx, li, du, dp, dq, scores, su, sp, sn):
    wid = lax.axis_index("s") * _NC + lax.axis_index("c")
    base = wid * _BPW

    pltpu.sync_copy(user_hbm.at[pl.ds(base, _BPW)], vidx.at[0])
    pltpu.sync_copy(pos_hbm.at[pl.ds(base, _BPW)], vidx.at[1])
    pltpu.sync_copy(neg_hbm.at[pl.ds(base, _BPW)], vidx.at[2])

    def lines(c, carry):
        off = c * _L
        for t in range(3):
            li[t, pl.ds(off, _L)] = vidx[t, pl.ds(off, _L)] >> 1
        return carry

    lax.fori_loop(0, _G, lines, 0)

    cu = pltpu.async_copy(tu_hbm.at[li.at[0]], du, su)
    cp = pltpu.async_copy(ti_hbm.at[li.at[1]], dp, sp)
    cn = pltpu.async_copy(ti_hbm.at[li.at[2]], dq, sn)
    cu.wait()
    cp.wait()
    cn.wait()

    lanes = lax.iota(jnp.int32, _L)

    def chunk(c, carry):
        off = c * _L
        k = lanes + jnp.int32(off)
        hu = (vidx[0, pl.ds(off, _L)] & 1) * _F
        hp = (vidx[1, pl.ds(off, _L)] & 1) * _F
        hq = (vidx[2, pl.ds(off, _L)] & 1) * _F
        acc = jnp.zeros((_L,), jnp.float32)
        for f in range(_F):
            u = plsc.load_gather(du, [k, hu + f])
            p = plsc.load_gather(dp, [k, hp + f])
            n = plsc.load_gather(dq, [k, hq + f])
            acc = acc + u * (p - n)
        scores[pl.ds(off, _L)] = 2.0 - 1.0 / (1.0 + jnp.exp(-acc))
        return carry

    lax.fori_loop(0, _G, chunk, 0)
    pltpu.sync_copy(scores, out_hbm.at[pl.ds(base, _BPW)])


@functools.lru_cache(maxsize=1)
def _build():
    mesh = plsc.VectorSubcoreMesh(
        core_axis_name="c", subcore_axis_name="s",
        num_cores=_NC, num_subcores=_NS)
    transpose = pl.kernel(
        _transpose_body,
        out_type=(jax.ShapeDtypeStruct((_NL, 2 * _F), jnp.float32),
                  jax.ShapeDtypeStruct((_NL, 2 * _F), jnp.float32)),
        mesh=mesh,
        scratch_types=[
            pltpu.VMEM((_F, _SLAB), jnp.float32),
            pltpu.VMEM((_SLAB // 2, 2 * _F), jnp.float32),
            pltpu.SemaphoreType.DMA,
            pltpu.SemaphoreType.DMA,
        ],
        compiler_params=pltpu.CompilerParams(needs_layout_passes=False),
    )
    gather = pl.kernel(
        _gather_body,
        out_type=jax.ShapeDtypeStruct((_B,), jnp.float32),
        mesh=mesh,
        scratch_types=[
            pltpu.VMEM((3, _BPW), jnp.int32),
            pltpu.VMEM((3, _BPW), jnp.int32),
            pltpu.VMEM((_BPW, 2 * _F), jnp.float32),
            pltpu.VMEM((_BPW, 2 * _F), jnp.float32),
            pltpu.VMEM((_BPW, 2 * _F), jnp.float32),
            pltpu.VMEM((_BPW,), jnp.float32),
            pltpu.SemaphoreType.DMA,
            pltpu.SemaphoreType.DMA,
            pltpu.SemaphoreType.DMA,
        ],
        compiler_params=pltpu.CompilerParams(needs_layout_passes=False),
    )
    return transpose, gather


@jax.jit
def kernel(user, posItem, negItem, user_W, item_W):
    transpose, gather = _build()
    tail_u = user_W[_NSLAB * _SLAB:, :].reshape(_TAIL_L, 2 * _F)
    tail_i = item_W[_NSLAB * _SLAB:, :].reshape(_TAIL_L, 2 * _F)
    tu, ti = transpose(user_W.T, item_W.T, tail_u, tail_i)
    out = gather(user, posItem, negItem, tu, ti)
    return out.reshape(-1, 1)


# speedup vs baseline: 2.4219x; 2.4219x over previous
import functools

import jax
import jax.numpy as jnp
from jax import lax
from jax.experimental import pallas as pl
from jax.experimental.pallas import tpu as pltpu
from jax.experimental.pallas import tpu_sc as plsc

_NC = 2
_NS = 16
_L = 16
_NW = _NC * _NS
_B = 4096
_F = 64
_N = 100000
_BPW = _B // _NW
_G = _BPW // _L

_SLAB = 128
_NSLAB = _N // _SLAB
_NL = _N // 2
_TAIL = _N - _NSLAB * _SLAB
_TAIL_L = _TAIL // 2
_TOT_SLABS = 2 * _NSLAB
_SPW = -(-_TOT_SLABS // _NW)


def _transpose_body(uwt_hbm, iwt_hbm, tail_u, tail_i, ou_hbm, oi_hbm,
                    slab, block, sem_in, sem_out):
    wid = lax.axis_index("s") * _NC + lax.axis_index("c")

    lanes = lax.iota(jnp.int32, _L)
    lin_perm = lanes >> 1
    col_perm = (lanes & 1) * jnp.int32(_F)

    def do_slab(k, carry):
        s = wid + k * _NW

        @pl.when(s < _TOT_SLABS)
        def _():
            is_item = s >= _NSLAB
            ib = jnp.where(is_item, s - _NSLAB, s)

            @pl.when(jnp.logical_not(is_item))
            def _():
                pltpu.async_copy(
                    uwt_hbm.at[:, pl.ds(ib * _SLAB, _SLAB)],
                    slab, sem_in).wait()

            @pl.when(is_item)
            def _():
                pltpu.async_copy(
                    iwt_hbm.at[:, pl.ds(ib * _SLAB, _SLAB)],
                    slab, sem_in).wait()

            for f in range(_F):
                for cb in range(_SLAB // _L):
                    v = slab[f, pl.ds(cb * _L, _L)]
                    lin = lin_perm + jnp.int32(cb * (_L // 2))
                    col = col_perm + jnp.int32(f)
                    plsc.store_scatter(block, [lin, col], v)

            @pl.when(jnp.logical_not(is_item))
            def _():
                pltpu.async_copy(
                    block,
                    ou_hbm.at[pl.ds(ib * (_SLAB // 2), _SLAB // 2)],
                    sem_out).wait()

            @pl.when(is_item)
            def _():
                pltpu.async_copy(
                    block,
                    oi_hbm.at[pl.ds(ib * (_SLAB // 2), _SLAB // 2)],
                    sem_out).wait()
        return carry

    lax.fori_loop(0, _SPW, do_slab, 0)

    @pl.when(wid == 0)
    def _():
        pltpu.sync_copy(tail_u, ou_hbm.at[pl.ds(_NL - _TAIL_L, _TAIL_L)])

    @pl.when(wid == 1)
    def _():
        pltpu.sync_copy(tail_i, oi_hbm.at[pl.ds(_NL - _TAIL_L, _TAIL_L)])


def _gather_body(user_hbm, pos_hbm, neg_hbm, tu_hbm, ti_hbm, out_hbm,
                 vidx, li, du, dp, dq, scores, su, sp, sn):
    wid = lax.axis_index("s") * _NC + lax.axis_index("c")
    base = wid * _BPW

    pltpu.sync_copy(user_hbm.at[pl.ds(base, _BPW)], vidx.at[0])
    pltpu.sync_copy(pos_hbm.at[pl.ds(base, _BPW)], vidx.at[1])
    pltpu.sync_copy(neg_hbm.at[pl.ds(base, _BPW)], vidx.at[2])

    def lines(c, carry):
        off = c * _L
        for t in range(3):
            li[t, pl.ds(off, _L)] = vidx[t, pl.ds(off, _L)] >> 1
        return carry

    lax.fori_loop(0, _G, lines, 0)

    cu = pltpu.async_copy(tu_hbm.at[li.at[0]], du, su)
    cp = pltpu.async_copy(ti_hbm.at[li.at[1]], dp, sp)
    cn = pltpu.async_copy(ti_hbm.at[li.at[2]], dq, sn)
    cu.wait()
    cp.wait()
    cn.wait()

    lanes = lax.iota(jnp.int32, _L)

    def chunk(c, carry):
        off = c * _L
        k = lanes + jnp.int32(off)
        hu = (vidx[0, pl.ds(off, _L)] & 1) * _F
        hp = (vidx[1, pl.ds(off, _L)] & 1) * _F
        hq = (vidx[2, pl.ds(off, _L)] & 1) * _F
        acc = jnp.zeros((_L,), jnp.float32)
        for f in range(_F):
            u = plsc.load_gather(du, [k, hu + f])
            p = plsc.load_gather(dp, [k, hp + f])
            n = plsc.load_gather(dq, [k, hq + f])
            acc = acc + u * (p - n)
        scores[pl.ds(off, _L)] = 2.0 - 1.0 / (1.0 + jnp.exp(-acc))
        return carry

    lax.fori_loop(0, _G, chunk, 0)
    pltpu.sync_copy(scores, out_hbm.at[pl.ds(base, _BPW)])


@functools.lru_cache(maxsize=1)
def _build():
    mesh = plsc.VectorSubcoreMesh(
        core_axis_name="c", subcore_axis_name="s",
        num_cores=_NC, num_subcores=_NS)
    transpose = pl.kernel(
        _transpose_body,
        out_type=(jax.ShapeDtypeStruct((_NL, 2 * _F), jnp.float32),
                  jax.ShapeDtypeStruct((_NL, 2 * _F), jnp.float32)),
        mesh=mesh,
        scratch_types=[
            pltpu.VMEM((_F, _SLAB), jnp.float32),
            pltpu.VMEM((_SLAB // 2, 2 * _F), jnp.float32),
            pltpu.SemaphoreType.DMA,
            pltpu.SemaphoreType.DMA,
        ],
        compiler_params=pltpu.CompilerParams(needs_layout_passes=False),
    )
    gather = pl.kernel(
        _gather_body,
        out_type=jax.ShapeDtypeStruct((_B,), jnp.float32),
        mesh=mesh,
        scratch_types=[
            pltpu.VMEM((3, _BPW), jnp.int32),
            pltpu.VMEM((3, _BPW), jnp.int32),
            pltpu.VMEM((_BPW, 2 * _F), jnp.float32),
            pltpu.VMEM((_BPW, 2 * _F), jnp.float32),
            pltpu.VMEM((_BPW, 2 * _F), jnp.float32),
            pltpu.VMEM((_BPW,), jnp.float32),
            pltpu.SemaphoreType.DMA,
            pltpu.SemaphoreType.DMA,
            pltpu.SemaphoreType.DMA,
        ],
        compiler_params=pltpu.CompilerParams(needs_layout_passes=False),
    )
    return transpose, gather


@jax.jit
def kernel(user, posItem, negItem, user_W, item_W):
    _, gather = _build()
    tu = user_W.reshape(_NL, 2 * _F)
    ti = item_W.reshape(_NL, 2 * _F)
    out = gather(user, posItem, negItem, tu, ti)
    return out.reshape(-1, 1)


# speedup vs baseline: 3.7217x; 1.5367x over previous
import functools

import jax
import jax.numpy as jnp
from jax import lax
from jax.experimental import pallas as pl
from jax.experimental.pallas import tpu as pltpu
from jax.experimental.pallas import tpu_sc as plsc

_NC = 2
_NS = 16
_L = 16
_NW = _NC * _NS
_B = 4096
_F = 64
_BPW = _B // _NW
_G = _BPW // _L


def _body(user_hbm, pos_hbm, neg_hbm, uw_hbm, iw_hbm, out_hbm,
          vidx, urows, prows, nrows, tbuf, scores,
          su, sp, sn):
    wid = lax.axis_index("s") * _NC + lax.axis_index("c")
    base = wid * _BPW

    pltpu.sync_copy(user_hbm.at[pl.ds(base, _BPW)], vidx.at[0])
    pltpu.sync_copy(pos_hbm.at[pl.ds(base, _BPW)], vidx.at[1])
    pltpu.sync_copy(neg_hbm.at[pl.ds(base, _BPW)], vidx.at[2])

    def fire(c, carry):
        vu = vidx[0, pl.ds(c * _L, _L)]
        vp = vidx[1, pl.ds(c * _L, _L)]
        vn = vidx[2, pl.ds(c * _L, _L)]
        for j in range(_L):
            r = c * _L + j
            pltpu.async_copy(uw_hbm.at[vu[j]], urows.at[r], su)
            pltpu.async_copy(iw_hbm.at[vp[j]], prows.at[r], sp)
            pltpu.async_copy(iw_hbm.at[vn[j]], nrows.at[r], sn)
        return carry

    lax.fori_loop(0, _G, fire, 0)
    pltpu.make_async_copy(uw_hbm.at[pl.ds(0, _BPW)], urows, su).wait()
    pltpu.make_async_copy(iw_hbm.at[pl.ds(0, _BPW)], prows, sp).wait()
    pltpu.make_async_copy(iw_hbm.at[pl.ds(0, _BPW)], nrows, sn).wait()

    lanes = lax.iota(jnp.int32, _L)

    def group(g, carry):
        for j in range(_L):
            r = g * _L + j
            acc = jnp.zeros((_L,), jnp.float32)
            for f in range(0, _F, _L):
                u = urows[r, pl.ds(f, _L)]
                p = prows[r, pl.ds(f, _L)]
                n = nrows[r, pl.ds(f, _L)]
                acc = acc + u * (p - n)
            tbuf[pl.ds(j * _L, _L)] = acc
        tot = jnp.zeros((_L,), jnp.float32)
        for c in range(_L):
            tot = tot + plsc.load_gather(tbuf, [lanes * _L + c])
        res = 2.0 - 1.0 / (1.0 + jnp.exp(-tot))
        scores[pl.ds(g * _L, _L)] = res
        return carry

    lax.fori_loop(0, _G, group, 0)
    pltpu.sync_copy(scores, out_hbm.at[pl.ds(base, _BPW)])


@functools.lru_cache(maxsize=1)
def _build():
    return pl.kernel(
        _body,
        out_type=jax.ShapeDtypeStruct((_B,), jnp.float32),
        mesh=plsc.VectorSubcoreMesh(
            core_axis_name="c", subcore_axis_name="s",
            num_cores=_NC, num_subcores=_NS),
        scratch_types=[
            pltpu.VMEM((3, _BPW), jnp.int32),
            pltpu.VMEM((_BPW, _F), jnp.float32),
            pltpu.VMEM((_BPW, _F), jnp.float32),
            pltpu.VMEM((_BPW, _F), jnp.float32),
            pltpu.VMEM((_L * _L,), jnp.float32),
            pltpu.VMEM((_BPW,), jnp.float32),
            pltpu.SemaphoreType.DMA,
            pltpu.SemaphoreType.DMA,
            pltpu.SemaphoreType.DMA,
        ],
        compiler_params=pltpu.CompilerParams(needs_layout_passes=False),
    )


@jax.jit
def kernel(user, posItem, negItem, user_W, item_W):
    out = _build()(user, posItem, negItem, user_W, item_W)
    return out.reshape(-1, 1)
